# baseline (speedup 1.0000x reference)
import jax
import jax.numpy as jnp
from jax import lax
from jax.experimental import pallas as pl
from jax.experimental.pallas import tpu as pltpu
from jax.experimental.pallas import tpu_sc as plsc

D_MODEL = 512
S_PAD = 2048
NC = 2
NS = 16
NW = NC * NS
NPW = S_PAD // NW
SBLK = 64


def _sc_pe_body(tab_h, idx_h, ope, iv, rows, sem):
    wid = lax.axis_index("s") * NC + lax.axis_index("c")
    pltpu.sync_copy(idx_h.at[pl.ds(wid * 3 * NPW, 3 * NPW)], iv)
    c0 = pltpu.async_copy(tab_h.at[iv.at[pl.ds(0, NPW)]], rows.at[0], sem)
    c1 = pltpu.async_copy(tab_h.at[iv.at[pl.ds(NPW, NPW)]], rows.at[1], sem)
    c2 = pltpu.async_copy(tab_h.at[iv.at[pl.ds(2 * NPW, NPW)]], rows.at[2], sem)
    c0.wait()
    c1.wait()
    c2.wait()

    def _row(r, carry):
        for j in range(D_MODEL // 16):
            sl = pl.ds(j * 16, 16)
            rows[0, r, sl] = rows[0, r, sl] + rows[1, r, sl] + rows[2, r, sl]
        return carry

    lax.fori_loop(0, NPW, _row, 0)
    pltpu.sync_copy(rows.at[0], ope.at[pl.ds(wid * NPW, NPW)])


def _sc_pe(tab, idx):
    mesh = plsc.VectorSubcoreMesh(core_axis_name="c", subcore_axis_name="s")
    f = pl.kernel(
        _sc_pe_body,
        mesh=mesh,
        out_type=jax.ShapeDtypeStruct((S_PAD, D_MODEL), jnp.float32),
        scratch_types=[
            pltpu.VMEM((3 * NPW,), jnp.int32),
            pltpu.VMEM((3, NPW, D_MODEL), jnp.float32),
            pltpu.SemaphoreType.DMA,
        ],
    )
    return f(tab, idx)


def _add_body(x_ref, pe_ref, o_ref):
    o_ref[...] = x_ref[...] + pe_ref[...][:, None, :]


def kernel(x, pos_x, pos_y, stab, token_to_x, token_to_y, token_to_stab):
    B, S, DM = x.shape
    pad = S_PAD - S
    nx = pos_x.shape[0]
    ix = jnp.pad(token_to_x[:S].astype(jnp.int32), (0, pad))
    iy = jnp.pad(token_to_y[:S].astype(jnp.int32), (0, pad))
    is_ = jnp.pad(token_to_stab[:S].astype(jnp.int32), (0, pad))
    tab = jnp.concatenate([pos_x, pos_y, stab], axis=0)
    idx3 = jnp.stack([ix, iy + nx, is_ + 2 * nx])
    idx = idx3.reshape(3, NW, NPW).transpose(1, 0, 2).reshape(-1)
    pe = _sc_pe(tab, idx)
    xt = jnp.transpose(x, (1, 0, 2))
    ns = pl.cdiv(S, SBLK)
    out_t = pl.pallas_call(
        _add_body,
        grid=(ns,),
        in_specs=[
            pl.BlockSpec((SBLK, B, DM), lambda s: (s, 0, 0)),
            pl.BlockSpec((SBLK, DM), lambda s: (s, 0)),
        ],
        out_specs=pl.BlockSpec((SBLK, B, DM), lambda s: (s, 0, 0)),
        out_shape=jax.ShapeDtypeStruct((S, B, DM), x.dtype),
    )(xt, pe)
    return jnp.transpose(out_t, (1, 0, 2))


# speedup vs baseline: 1.0084x; 1.0084x over previous
import jax
import jax.numpy as jnp
from jax import lax
from jax.experimental import pallas as pl
from jax.experimental.pallas import tpu as pltpu
from jax.experimental.pallas import tpu_sc as plsc

D_MODEL = 512
S_PAD = 2048
NC = 2
NS = 16
NW = NC * NS
NPW = S_PAD // NW
SBLK = 128


def _sc_pe_body(tab_h, idx_h, ope, iv, rows, sem):
    wid = lax.axis_index("s") * NC + lax.axis_index("c")
    pltpu.sync_copy(idx_h.at[pl.ds(wid * 3 * NPW, 3 * NPW)], iv)
    c0 = pltpu.async_copy(tab_h.at[iv.at[pl.ds(0, NPW)]], rows.at[0], sem)
    c1 = pltpu.async_copy(tab_h.at[iv.at[pl.ds(NPW, NPW)]], rows.at[1], sem)
    c2 = pltpu.async_copy(tab_h.at[iv.at[pl.ds(2 * NPW, NPW)]], rows.at[2], sem)
    c0.wait()
    c1.wait()
    c2.wait()

    def _row(r, carry):
        for j in range(D_MODEL // 16):
            sl = pl.ds(j * 16, 16)
            rows[0, r, sl] = rows[0, r, sl] + rows[1, r, sl] + rows[2, r, sl]
        return carry

    lax.fori_loop(0, NPW, _row, 0)
    pltpu.sync_copy(rows.at[0], ope.at[pl.ds(wid * NPW, NPW)])


def _sc_pe(tab, idx):
    mesh = plsc.VectorSubcoreMesh(core_axis_name="c", subcore_axis_name="s")
    f = pl.kernel(
        _sc_pe_body,
        mesh=mesh,
        out_type=jax.ShapeDtypeStruct((S_PAD, D_MODEL), jnp.float32),
        scratch_types=[
            pltpu.VMEM((3 * NPW,), jnp.int32),
            pltpu.VMEM((3, NPW, D_MODEL), jnp.float32),
            pltpu.SemaphoreType.DMA,
        ],
    )
    return f(tab, idx)


def _add_body(x_ref, pe_ref, o_ref):
    o_ref[...] = x_ref[...] + pe_ref[...][:, None, :]


def kernel(x, pos_x, pos_y, stab, token_to_x, token_to_y, token_to_stab):
    B, S, DM = x.shape
    pad = S_PAD - S
    nx = pos_x.shape[0]
    ix = jnp.pad(token_to_x[:S].astype(jnp.int32), (0, pad))
    iy = jnp.pad(token_to_y[:S].astype(jnp.int32), (0, pad))
    is_ = jnp.pad(token_to_stab[:S].astype(jnp.int32), (0, pad))
    tab = jnp.concatenate([pos_x, pos_y, stab], axis=0)
    idx3 = jnp.stack([ix, iy + nx, is_ + 2 * nx])
    idx = idx3.reshape(3, NW, NPW).transpose(1, 0, 2).reshape(-1)
    pe = _sc_pe(tab, idx)
    xt = jnp.transpose(x, (1, 0, 2))
    ns = pl.cdiv(S, SBLK)
    out_t = pl.pallas_call(
        _add_body,
        grid=(ns,),
        in_specs=[
            pl.BlockSpec((SBLK, B, DM), lambda s: (s, 0, 0)),
            pl.BlockSpec((SBLK, DM), lambda s: (s, 0)),
        ],
        out_specs=pl.BlockSpec((SBLK, B, DM), lambda s: (s, 0, 0)),
        out_shape=jax.ShapeDtypeStruct((S, B, DM), x.dtype),
    )(xt, pe)
    return jnp.transpose(out_t, (1, 0, 2))


# speedup vs baseline: 1.2265x; 1.2162x over previous
import jax
import jax.numpy as jnp
from jax import lax
from jax.experimental import pallas as pl
from jax.experimental.pallas import tpu as pltpu
from jax.experimental.pallas import tpu_sc as plsc

D_MODEL = 512
NC = 2
NS = 16
NW = NC * NS
SBLK = 64
S1 = 1536
NB1 = S1 // SBLK
TPAD = 512
NPW = TPAD // NW
NTP = 256
LC = D_MODEL // 16


def _sc_pe_body(tab_h, idx_h, ope, iv, rows, sem):
    wid = lax.axis_index("s") * NC + lax.axis_index("c")
    pltpu.sync_copy(idx_h.at[pl.ds(wid * 3 * NPW, 3 * NPW)], iv)
    c0 = pltpu.async_copy(tab_h.at[iv.at[pl.ds(0, NPW)]], rows.at[0], sem)
    c1 = pltpu.async_copy(tab_h.at[iv.at[pl.ds(NPW, NPW)]], rows.at[1], sem)
    c2 = pltpu.async_copy(tab_h.at[iv.at[pl.ds(2 * NPW, NPW)]], rows.at[2], sem)
    c0.wait()
    c1.wait()
    c2.wait()

    def _row(r, carry):
        for j in range(LC):
            sl = pl.ds(j * 16, 16)
            rows[0, r, sl] = rows[0, r, sl] + rows[1, r, sl] + rows[2, r, sl]
        return carry

    lax.fori_loop(0, NPW, _row, 0)
    pltpu.sync_copy(rows.at[0], ope.at[pl.ds(wid * NPW, NPW)])


def _sc_pe(tab, idx):
    mesh = plsc.VectorSubcoreMesh(core_axis_name="c", subcore_axis_name="s")
    f = pl.kernel(
        _sc_pe_body,
        mesh=mesh,
        out_type=jax.ShapeDtypeStruct((TPAD, D_MODEL), jnp.float32),
        scratch_types=[
            pltpu.VMEM((3 * NPW,), jnp.int32),
            pltpu.VMEM((3, NPW, D_MODEL), jnp.float32),
            pltpu.SemaphoreType.DMA,
        ],
    )
    return f(tab, idx)


def _tc1_body(x_ref, idx_ref, tab_ref, o_ref):
    r = lax.broadcasted_iota(jnp.int32, (SBLK, NTP), 1)
    m = ((idx_ref[0, 0, :][:, None] == r).astype(jnp.float32)
         + (idx_ref[0, 1, :][:, None] == r).astype(jnp.float32)
         + (idx_ref[0, 2, :][:, None] == r).astype(jnp.float32))
    pe = jnp.dot(m, tab_ref[...], preferred_element_type=jnp.float32,
                 precision=lax.Precision.HIGHEST)
    o_ref[...] = x_ref[...] + pe[:, None, :]


def _tc2_body(x_ref, pe_ref, o1_ref, o_ref):
    del o1_ref
    o_ref[...] = x_ref[...] + pe_ref[...][:, None, :]


def kernel(x, pos_x, pos_y, stab, token_to_x, token_to_y, token_to_stab):
    B, S, DM = x.shape
    nx = pos_x.shape[0]
    tpad = S1 + TPAD - S
    ix = jnp.pad(token_to_x[:S].astype(jnp.int32), (0, tpad))
    iy = jnp.pad(token_to_y[:S].astype(jnp.int32), (0, tpad)) + nx
    is_ = jnp.pad(token_to_stab[:S].astype(jnp.int32), (0, tpad)) + 2 * nx
    tab = jnp.concatenate([pos_x, pos_y, stab], axis=0)
    tabp = jnp.pad(tab, ((0, NTP - tab.shape[0]), (0, 0)))

    idx3_t = jnp.stack([ix[S1:], iy[S1:], is_[S1:]])
    idx_t = idx3_t.reshape(3, NW, NPW).transpose(1, 0, 2).reshape(-1)
    pe_tail = _sc_pe(tabp, idx_t)

    idx_head = jnp.stack(
        [ix[:S1].reshape(NB1, SBLK),
         iy[:S1].reshape(NB1, SBLK),
         is_[:S1].reshape(NB1, SBLK)], axis=1)
    xt = jnp.transpose(x, (1, 0, 2))

    o1 = pl.pallas_call(
        _tc1_body,
        grid=(NB1,),
        in_specs=[
            pl.BlockSpec((SBLK, B, DM), lambda s: (s, 0, 0)),
            pl.BlockSpec((1, 3, SBLK), lambda s: (s, 0, 0)),
            pl.BlockSpec((NTP, DM), lambda s: (0, 0)),
        ],
        out_specs=pl.BlockSpec((SBLK, B, DM), lambda s: (s, 0, 0)),
        out_shape=jax.ShapeDtypeStruct((S, B, DM), x.dtype),
    )(xt, idx_head, tabp)

    nb2 = pl.cdiv(S - S1, SBLK)
    out_t = pl.pallas_call(
        _tc2_body,
        grid=(nb2,),
        in_specs=[
            pl.BlockSpec((SBLK, B, DM), lambda s: (s + NB1, 0, 0)),
            pl.BlockSpec((SBLK, DM), lambda s: (s, 0)),
            pl.BlockSpec(memory_space=pl.ANY),
        ],
        out_specs=pl.BlockSpec((SBLK, B, DM), lambda s: (s + NB1, 0, 0)),
        out_shape=jax.ShapeDtypeStruct((S, B, DM), x.dtype),
        input_output_aliases={2: 0},
    )(xt, pe_tail, o1)
    return jnp.transpose(out_t, (1, 0, 2))
